# baseline (speedup 1.0000x reference)
import numpy as np
import jax
import jax.numpy as jnp
from jax import lax
from jax.experimental import pallas as pl
from jax.experimental.pallas import tpu as pltpu

HID = 128
EPS = 1e-5
N_SUBJECTS = 4
T = 281
CIN = 271
NC = 1854

NB1 = 32
TP1 = 288
TCOMP = 43
TPAD = 48
NB2 = 128

_INV_SQRT2 = 0.7071067811865476


def _gelu(x):
    return 0.5 * x * (1.0 + lax.erf(x * _INV_SQRT2))


def _shift_add(dcat):
    zrow = jnp.zeros((1, 128), jnp.float32)
    return (dcat[:, 128:256]
            + jnp.concatenate([zrow, dcat[:-1, 0:128]], axis=0)
            + jnp.concatenate([dcat[1:, 256:384], zrow], axis=0))


def _conv_tc(z, wcat):
    dcat = jnp.dot(z.astype(jnp.bfloat16), wcat,
                   preferred_element_type=jnp.float32)
    return _shift_add(dcat)


def _p1_kernel(x_ref, w0k_ref, wk_ref, rows_ref, out_ref):
    def bn(k):
        return (jnp.tile(rows_ref[2 * k], (NB1, 1)),
                jnp.tile(rows_ref[2 * k + 1], (NB1, 1)))

    zpad = jnp.zeros((TP1 - T, CIN), jnp.bfloat16)
    pieces = []
    for b in range(NB1):
        pieces.append(x_ref[b].astype(jnp.bfloat16))
        pieces.append(zpad)
    x = jnp.concatenate(pieces, axis=0)
    dcat = jnp.dot(x, w0k_ref[...], preferred_element_type=jnp.float32)
    y = _shift_add(dcat)
    s, o = bn(0)
    a = _gelu(y * s + o)
    y = _conv_tc(a, wk_ref[0]) + a
    s, o = bn(1)
    h = _gelu(y * s + o)
    y = _conv_tc(h, wk_ref[1]) + h
    s, o = bn(2)
    a = _gelu(y * s + o)
    y = _conv_tc(a, wk_ref[2]) + a
    s, o = bn(3)
    h = _gelu(y * s + o)
    out_ref[...] = h.reshape(NB1, TP1, 128).sum(axis=1)


def _p2_kernel(xsum_ref, subjf_ref, w2_ref, hk_ref, rows_ref,
               wts_ref, wcls_ref, bcls_ref, out_ref):
    xs = xsum_ref[...]
    bo = rows_ref[8]

    def seg(s, n):
        col = jnp.dot(xs, w2_ref[s], preferred_element_type=jnp.float32) + bo
        return jnp.broadcast_to(col[:, None, :], (NB2, n, 128))

    pieces = [seg(0, 9), seg(1, 1), seg(2, 9), seg(3, 1),
              seg(4, 9), seg(5, 1), seg(6, 9)]
    sf = subjf_ref[...]
    for u in range(N_SUBJECTS):
        oh = jnp.where(sf == float(u), 1.0, 0.0)
        pieces.append(oh[:, None, :])
    pieces.append(jnp.zeros((NB2, TPAD - TCOMP, 128), jnp.float32))
    z = jnp.concatenate(pieces, axis=1)
    zf = z.reshape(NB2 * TPAD, 128)

    mask = jnp.tile(jnp.where(wts_ref[...] > 0.0, 1.0, 0.0), (NB2, 1))

    for c in range(2):
        y = _conv_tc(zf, hk_ref[2 * c]) + zf
        a = _gelu(y * rows_ref[4 * c + 0] + rows_ref[4 * c + 1]) * mask
        y = _conv_tc(a, hk_ref[2 * c + 1]) + a
        zf = _gelu(y * rows_ref[4 * c + 2] + rows_ref[4 * c + 3]) * mask

    zw = zf * jnp.tile(wts_ref[...], (NB2, 1))
    pooled = zw.reshape(NB2, TPAD, 128).sum(axis=1) * (1.0 / 285.0)
    out_ref[...] = (jnp.dot(pooled, wcls_ref[...],
                            preferred_element_type=jnp.float32)
                    + bcls_ref[...])


def _stage_rows(p, which, bias):
    bn = p[which]
    s = bn['gamma'] / jnp.sqrt(bn['var'] + EPS)
    return [s, bn['beta'] - bn['mean'] * s + s * bias]


def _block_rows(p):
    return (_stage_rows(p, 'bn0', p['b0']) + _stage_rows(p, 'bn1', p['b1']))


def _seg_fold_consts():
    hfun = [lambda c: 0, lambda c: 0 if c < 32 else 1, lambda c: 1,
            lambda c: 1 if c < 64 else 2, lambda c: 2,
            lambda c: 2 if c < 96 else 3, lambda c: 3]
    S = np.zeros((7 * 128, 128), np.float32)
    for s, h in enumerate(hfun):
        for c in range(128):
            S[s * 128 + 32 * h(c) + (c % 32), c] += 1.0
    return S


_SEG_S = _seg_fold_consts()

_WTS = []
for _L in (70, 1, 69, 1, 69, 1, 70, 1, 1, 1, 1):
    _WTS += ([1.0] * 4 + [float(_L - 8)] + [1.0] * 4) if _L >= 9 else [1.0] * _L
_WTS += [0.0] * (TPAD - TCOMP)
_WTS = np.asarray(_WTS, np.float32)


def kernel(X, subject_idxs, params):
    B = X.shape[0]
    p = params
    bf = jnp.bfloat16

    def _wcat(w):
        return w.transpose(1, 2, 0).reshape(w.shape[1], 3 * w.shape[0])

    w0k = _wcat(p['block0']['w0']).astype(bf)
    wk = jnp.stack([p['block0']['w1'], p['block1']['w0'],
                    p['block1']['w1']]).transpose(0, 2, 3, 1) \
        .reshape(3, 128, 384).astype(bf)
    rows1 = jnp.stack(_block_rows(p['block0']) + _block_rows(p['block1']))
    live = (jnp.arange(TP1) < T).astype(jnp.float32)[None, :, None]
    rows1f = jnp.broadcast_to(rows1[:, None, :], (8, TP1, 128)) * live

    sw = (jnp.asarray(_SEG_S) @ p['wo']).reshape(7, 128, 128)
    w2 = jnp.einsum('ij,sjk->sik', p['wv'], sw)
    hk = jnp.stack([p['head0']['w0'], p['head0']['w1'],
                    p['head1']['w0'], p['head1']['w1']]).transpose(0, 2, 3, 1) \
        .reshape(4, 128, 384).astype(bf)
    rows2 = jnp.stack(_block_rows(p['head0']) + _block_rows(p['head1'])
                      + [p['bo']])
    wts = jnp.broadcast_to(jnp.asarray(_WTS)[:, None], (TPAD, 128))
    subjf = jnp.broadcast_to(subject_idxs.astype(jnp.float32)[:, None],
                             (B, 128))
    bcls = p['bcls'][None, :]

    Xb = jnp.transpose(X, (0, 2, 1))

    xsum = pl.pallas_call(
        _p1_kernel,
        grid=(B // NB1,),
        in_specs=[
            pl.BlockSpec((NB1, T, CIN), lambda i: (i, 0, 0)),
            pl.BlockSpec((CIN, 384), lambda i: (0, 0)),
            pl.BlockSpec((3, 128, 384), lambda i: (0, 0, 0)),
            pl.BlockSpec((8, TP1, 128), lambda i: (0, 0, 0)),
        ],
        out_specs=pl.BlockSpec((NB1, 128), lambda i: (i, 0)),
        out_shape=jax.ShapeDtypeStruct((B, 128), jnp.float32),
        compiler_params=pltpu.CompilerParams(
            dimension_semantics=("arbitrary",),
            vmem_limit_bytes=100 * 1024 * 1024,
        ),
        name="scband_p1_convs",
    )(Xb, w0k, wk, rows1f)

    out = pl.pallas_call(
        _p2_kernel,
        grid=(B // NB2,),
        in_specs=[
            pl.BlockSpec((NB2, 128), lambda i: (i, 0)),
            pl.BlockSpec((NB2, 128), lambda i: (i, 0)),
            pl.BlockSpec((7, 128, 128), lambda i: (0, 0, 0)),
            pl.BlockSpec((4, 128, 384), lambda i: (0, 0, 0)),
            pl.BlockSpec((9, 128), lambda i: (0, 0)),
            pl.BlockSpec((TPAD, 128), lambda i: (0, 0)),
            pl.BlockSpec((128, NC), lambda i: (0, 0)),
            pl.BlockSpec((1, NC), lambda i: (0, 0)),
        ],
        out_specs=pl.BlockSpec((NB2, NC), lambda i: (i, 0)),
        out_shape=jax.ShapeDtypeStruct((B, NC), jnp.float32),
        compiler_params=pltpu.CompilerParams(
            dimension_semantics=("arbitrary",),
            vmem_limit_bytes=100 * 1024 * 1024,
        ),
        name="scband_p2_head",
    )(xsum, subjf, w2, hk, rows2, wts, p['wcls'], bcls)

    return out


# speedup vs baseline: 1.0049x; 1.0049x over previous
import numpy as np
import jax
import jax.numpy as jnp
from jax import lax
from jax.experimental import pallas as pl
from jax.experimental.pallas import tpu as pltpu

HID = 128
EPS = 1e-5
N_SUBJECTS = 4
T = 281
CIN = 271
NC = 1854

NB1 = 16
TP1 = 288
TCOMP = 43
TPAD = 48
NB2 = 128

_INV_SQRT2 = 0.7071067811865476


def _gelu(x):
    return 0.5 * x * (1.0 + lax.erf(x * _INV_SQRT2))


def _shift_add(dcat):
    zrow = jnp.zeros((1, 128), jnp.float32)
    return (dcat[:, 128:256]
            + jnp.concatenate([zrow, dcat[:-1, 0:128]], axis=0)
            + jnp.concatenate([dcat[1:, 256:384], zrow], axis=0))


def _conv_tc(z, wcat):
    dcat = jnp.dot(z.astype(jnp.bfloat16), wcat,
                   preferred_element_type=jnp.float32)
    return _shift_add(dcat)


def _p1_kernel(x_ref, w0k_ref, wk_ref, rows_ref, out_ref):
    def bn(k):
        return (jnp.tile(rows_ref[2 * k], (NB1, 1)),
                jnp.tile(rows_ref[2 * k + 1], (NB1, 1)))

    zpad = jnp.zeros((TP1 - T, CIN), jnp.bfloat16)
    pieces = []
    for b in range(NB1):
        pieces.append(x_ref[b].astype(jnp.bfloat16))
        pieces.append(zpad)
    x = jnp.concatenate(pieces, axis=0)
    dcat = jnp.dot(x, w0k_ref[...], preferred_element_type=jnp.float32)
    y = _shift_add(dcat)
    s, o = bn(0)
    a = _gelu(y * s + o)
    y = _conv_tc(a, wk_ref[0]) + a
    s, o = bn(1)
    h = _gelu(y * s + o)
    y = _conv_tc(h, wk_ref[1]) + h
    s, o = bn(2)
    a = _gelu(y * s + o)
    y = _conv_tc(a, wk_ref[2]) + a
    s, o = bn(3)
    h = _gelu(y * s + o)
    out_ref[...] = h.reshape(NB1, TP1, 128).sum(axis=1)


def _p2_kernel(xsum_ref, subjf_ref, w2_ref, hk_ref, rows_ref,
               wts_ref, wcls_ref, bcls_ref, out_ref):
    xs = xsum_ref[...]
    bo = rows_ref[8]

    def seg(s, n):
        col = jnp.dot(xs, w2_ref[s], preferred_element_type=jnp.float32) + bo
        return jnp.broadcast_to(col[:, None, :], (NB2, n, 128))

    pieces = [seg(0, 9), seg(1, 1), seg(2, 9), seg(3, 1),
              seg(4, 9), seg(5, 1), seg(6, 9)]
    sf = subjf_ref[...]
    for u in range(N_SUBJECTS):
        oh = jnp.where(sf == float(u), 1.0, 0.0)
        pieces.append(oh[:, None, :])
    pieces.append(jnp.zeros((NB2, TPAD - TCOMP, 128), jnp.float32))
    z = jnp.concatenate(pieces, axis=1)
    zf = z.reshape(NB2 * TPAD, 128)

    mask = jnp.tile(jnp.where(wts_ref[...] > 0.0, 1.0, 0.0), (NB2, 1))

    for c in range(2):
        y = _conv_tc(zf, hk_ref[2 * c]) + zf
        a = _gelu(y * rows_ref[4 * c + 0] + rows_ref[4 * c + 1]) * mask
        y = _conv_tc(a, hk_ref[2 * c + 1]) + a
        zf = _gelu(y * rows_ref[4 * c + 2] + rows_ref[4 * c + 3]) * mask

    zw = zf * jnp.tile(wts_ref[...], (NB2, 1))
    pooled = zw.reshape(NB2, TPAD, 128).sum(axis=1) * (1.0 / 285.0)
    out_ref[...] = (jnp.dot(pooled, wcls_ref[...],
                            preferred_element_type=jnp.float32)
                    + bcls_ref[...])


def _stage_rows(p, which, bias):
    bn = p[which]
    s = bn['gamma'] / jnp.sqrt(bn['var'] + EPS)
    return [s, bn['beta'] - bn['mean'] * s + s * bias]


def _block_rows(p):
    return (_stage_rows(p, 'bn0', p['b0']) + _stage_rows(p, 'bn1', p['b1']))


def _seg_fold_consts():
    hfun = [lambda c: 0, lambda c: 0 if c < 32 else 1, lambda c: 1,
            lambda c: 1 if c < 64 else 2, lambda c: 2,
            lambda c: 2 if c < 96 else 3, lambda c: 3]
    S = np.zeros((7 * 128, 128), np.float32)
    for s, h in enumerate(hfun):
        for c in range(128):
            S[s * 128 + 32 * h(c) + (c % 32), c] += 1.0
    return S


_SEG_S = _seg_fold_consts()

_WTS = []
for _L in (70, 1, 69, 1, 69, 1, 70, 1, 1, 1, 1):
    _WTS += ([1.0] * 4 + [float(_L - 8)] + [1.0] * 4) if _L >= 9 else [1.0] * _L
_WTS += [0.0] * (TPAD - TCOMP)
_WTS = np.asarray(_WTS, np.float32)


def kernel(X, subject_idxs, params):
    B = X.shape[0]
    p = params
    bf = jnp.bfloat16

    def _wcat(w):
        return w.transpose(1, 2, 0).reshape(w.shape[1], 3 * w.shape[0])

    w0k = _wcat(p['block0']['w0']).astype(bf)
    wk = jnp.stack([p['block0']['w1'], p['block1']['w0'],
                    p['block1']['w1']]).transpose(0, 2, 3, 1) \
        .reshape(3, 128, 384).astype(bf)
    rows1 = jnp.stack(_block_rows(p['block0']) + _block_rows(p['block1']))
    live = (jnp.arange(TP1) < T).astype(jnp.float32)[None, :, None]
    rows1f = jnp.broadcast_to(rows1[:, None, :], (8, TP1, 128)) * live

    sw = (jnp.asarray(_SEG_S) @ p['wo']).reshape(7, 128, 128)
    w2 = jnp.einsum('ij,sjk->sik', p['wv'], sw)
    hk = jnp.stack([p['head0']['w0'], p['head0']['w1'],
                    p['head1']['w0'], p['head1']['w1']]).transpose(0, 2, 3, 1) \
        .reshape(4, 128, 384).astype(bf)
    rows2 = jnp.stack(_block_rows(p['head0']) + _block_rows(p['head1'])
                      + [p['bo']])
    wts = jnp.broadcast_to(jnp.asarray(_WTS)[:, None], (TPAD, 128))
    subjf = jnp.broadcast_to(subject_idxs.astype(jnp.float32)[:, None],
                             (B, 128))
    bcls = p['bcls'][None, :]

    Xb = jnp.transpose(X, (0, 2, 1))

    xsum = pl.pallas_call(
        _p1_kernel,
        grid=(B // NB1,),
        in_specs=[
            pl.BlockSpec((NB1, T, CIN), lambda i: (i, 0, 0)),
            pl.BlockSpec((CIN, 384), lambda i: (0, 0)),
            pl.BlockSpec((3, 128, 384), lambda i: (0, 0, 0)),
            pl.BlockSpec((8, TP1, 128), lambda i: (0, 0, 0)),
        ],
        out_specs=pl.BlockSpec((NB1, 128), lambda i: (i, 0)),
        out_shape=jax.ShapeDtypeStruct((B, 128), jnp.float32),
        compiler_params=pltpu.CompilerParams(
            dimension_semantics=("arbitrary",),
            vmem_limit_bytes=100 * 1024 * 1024,
        ),
        name="scband_p1_convs",
    )(Xb, w0k, wk, rows1f)

    out = pl.pallas_call(
        _p2_kernel,
        grid=(B // NB2,),
        in_specs=[
            pl.BlockSpec((NB2, 128), lambda i: (i, 0)),
            pl.BlockSpec((NB2, 128), lambda i: (i, 0)),
            pl.BlockSpec((7, 128, 128), lambda i: (0, 0, 0)),
            pl.BlockSpec((4, 128, 384), lambda i: (0, 0, 0)),
            pl.BlockSpec((9, 128), lambda i: (0, 0)),
            pl.BlockSpec((TPAD, 128), lambda i: (0, 0)),
            pl.BlockSpec((128, NC), lambda i: (0, 0)),
            pl.BlockSpec((1, NC), lambda i: (0, 0)),
        ],
        out_specs=pl.BlockSpec((NB2, NC), lambda i: (i, 0)),
        out_shape=jax.ShapeDtypeStruct((B, NC), jnp.float32),
        compiler_params=pltpu.CompilerParams(
            dimension_semantics=("arbitrary",),
            vmem_limit_bytes=100 * 1024 * 1024,
        ),
        name="scband_p2_head",
    )(xsum, subjf, w2, hk, rows2, wts, p['wcls'], bcls)

    return out
